# baseline (speedup 1.0000x reference)
import functools

import jax
import jax.numpy as jnp
from jax import lax
from jax.experimental import pallas as pl
from jax.experimental.pallas import tpu as pltpu
from jax.experimental.pallas import tpu_sc as plsc

B, H, T, DH = 2, 16, 4096, 64
BUCKETS = 64
BSZ = T // BUCKETS
BH = B * H
SCALE = 1024.0 ** -0.5
L = 16


def _r_kernel(q_ref, k_ref, w_ref, r_ref):
    qs = jnp.sum(q_ref[0, 0].reshape(BUCKETS, BSZ, DH), axis=1)
    ks = jnp.sum(k_ref[0, 0].reshape(BUCKETS, BSZ, DH), axis=1)
    x = jnp.concatenate([qs, ks], axis=1)
    r = jnp.dot(x, w_ref[0, 0], preferred_element_type=jnp.float32)
    r = jnp.maximum(r, 0.0)
    r_ref[0] = jax.nn.softmax(r, axis=-1).T


def _sc_top1(r_hbm, idx_hbm, val_hbm, rv, iv, vv, sem):
    wid = lax.axis_index("c") * 16 + lax.axis_index("s")
    pltpu.async_copy(r_hbm.at[wid], rv, sem).wait()
    for grp in range(BUCKETS // L):
        bestv = jnp.zeros((L,), jnp.int32)
        mxv = jnp.full((L,), -1.0, jnp.float32)
        for s in range(BUCKETS):
            v = rv[s, pl.ds(grp * L, L)]
            upd = v > mxv
            bestv = jnp.where(upd, s, bestv)
            mxv = jnp.where(upd, v, mxv)
        iv[pl.ds(grp * L, L)] = bestv
        vv[pl.ds(grp * L, L)] = mxv
    cp0 = pltpu.async_copy(iv, idx_hbm.at[wid], sem)
    cp0.wait()
    cp1 = pltpu.async_copy(vv, val_hbm.at[wid], sem)
    cp1.wait()


def _attn_kernel(idx_sref, val_sref, q_ref, k_ref, v_ref, o_ref,
                 kcat_ref, vcat_ref, p_ref):
    i = pl.program_id(0)
    ka = k_ref[0, 0].astype(jnp.bfloat16).reshape(BUCKETS, BSZ, DH)
    va = v_ref[0, 0].astype(jnp.bfloat16).reshape(BUCKETS, BSZ, DH)
    kcat_ref[:, BSZ:, :] = ka
    vcat_ref[:, BSZ:, :DH] = va
    vcat_ref[:, :, DH:] = jnp.ones((BUCKETS, 2 * BSZ, DH), jnp.bfloat16)
    for u in range(BUCKETS):
        g = idx_sref[i, u]
        s = val_sref[i, u]
        kcat_ref[u, :BSZ, :] = (
            k_ref[0, 0, pl.ds(g * BSZ, BSZ), :] * s).astype(jnp.bfloat16)
        vcat_ref[u, :BSZ, :DH] = (
            v_ref[0, 0, pl.ds(g * BSZ, BSZ), :] * s).astype(jnp.bfloat16)
    qa = (q_ref[0, 0] * SCALE).astype(jnp.bfloat16)
    for u in range(BUCKETS):
        d = jax.lax.dot_general(
            qa[u * BSZ:(u + 1) * BSZ, :], kcat_ref[u],
            (((1,), (1,)), ((), ())), preferred_element_type=jnp.float32)
        p_ref[u] = jnp.exp(d).astype(jnp.bfloat16)
    for u in range(BUCKETS):
        acc = jax.lax.dot_general(
            p_ref[u], vcat_ref[u], (((1,), (0,)), ((), ())),
            preferred_element_type=jnp.float32)
        o_ref[0, u] = acc[:, :DH] / acc[:, DH:DH + 1]


@jax.jit
def kernel(q, k, v, W):
    r = pl.pallas_call(
        _r_kernel,
        grid=(BH,),
        in_specs=[
            pl.BlockSpec((1, 1, T, DH), lambda i: (i // H, jax.lax.rem(i, H), 0, 0)),
            pl.BlockSpec((1, 1, T, DH), lambda i: (i // H, jax.lax.rem(i, H), 0, 0)),
            pl.BlockSpec((1, 1, 2 * DH, BUCKETS),
                         lambda i: (0, jax.lax.rem(i, H), 0, 0)),
        ],
        out_specs=pl.BlockSpec((1, BUCKETS, BUCKETS), lambda i: (i, 0, 0)),
        out_shape=jax.ShapeDtypeStruct((BH, BUCKETS, BUCKETS), jnp.float32),
    )(q, k, W)

    mesh = plsc.VectorSubcoreMesh(core_axis_name="c", subcore_axis_name="s")
    idx, val = pl.kernel(
        _sc_top1,
        mesh=mesh,
        out_type=[
            jax.ShapeDtypeStruct((BH, BUCKETS), jnp.int32),
            jax.ShapeDtypeStruct((BH, BUCKETS), jnp.float32),
        ],
        scratch_types=[
            pltpu.VMEM((BUCKETS, BUCKETS), jnp.float32),
            pltpu.VMEM((BUCKETS,), jnp.int32),
            pltpu.VMEM((BUCKETS,), jnp.float32),
            pltpu.SemaphoreType.DMA,
        ],
    )(r)

    out = pl.pallas_call(
        _attn_kernel,
        grid_spec=pltpu.PrefetchScalarGridSpec(
            num_scalar_prefetch=2,
            grid=(BH,),
            in_specs=[
                pl.BlockSpec((1, 1, T, DH),
                             lambda i, *_: (i // H, jax.lax.rem(i, H), 0, 0)),
                pl.BlockSpec((1, 1, T, DH),
                             lambda i, *_: (i // H, jax.lax.rem(i, H), 0, 0)),
                pl.BlockSpec((1, 1, T, DH),
                             lambda i, *_: (i // H, jax.lax.rem(i, H), 0, 0)),
            ],
            out_specs=pl.BlockSpec(
                (1, BUCKETS, BSZ, DH), lambda i, *_: (i, 0, 0, 0)),
            scratch_shapes=[
                pltpu.VMEM((BUCKETS, 2 * BSZ, DH), jnp.bfloat16),
                pltpu.VMEM((BUCKETS, 2 * BSZ, 2 * DH), jnp.bfloat16),
                pltpu.VMEM((BUCKETS, BSZ, 2 * BSZ), jnp.bfloat16),
            ],
        ),
        out_shape=jax.ShapeDtypeStruct((BH, BUCKETS, BSZ, DH), jnp.float32),
    )(idx, val, q, k, v)
    return out


# speedup vs baseline: 1.2172x; 1.2172x over previous
import jax
import jax.numpy as jnp
from jax.experimental import pallas as pl
from jax.experimental.pallas import tpu as pltpu

B, H, T, DH = 2, 16, 4096, 64
BUCKETS = 64
BSZ = T // BUCKETS
BH = B * H
G = 2
SCALE = 1024.0 ** -0.5


def _sinkhorn_kernel(q_ref, k_ref, v_ref, w_ref, o_ref,
                     kcat_ref, vcat_ref, p_ref,
                     iv_ref, vv_ref, idx_smem, val_smem, sem0, sem1):
    for hh in range(G):
        qs = jnp.sum(q_ref[0, hh].reshape(BUCKETS, BSZ, DH), axis=1)
        ks = jnp.sum(k_ref[0, hh].reshape(BUCKETS, BSZ, DH), axis=1)
        x = jnp.concatenate([qs, ks], axis=1)
        r = jnp.dot(x, w_ref[0, hh], preferred_element_type=jnp.float32)
        r = jnp.maximum(r, 0.0)
        r = jax.nn.softmax(r, axis=-1)
        rt = r.T
        m = jnp.max(rt, axis=0, keepdims=True)
        row = jax.lax.broadcasted_iota(jnp.int32, (BUCKETS, BUCKETS), 0)
        idx = jnp.min(jnp.where(rt >= m, row, BUCKETS), axis=0, keepdims=True)
        iv_ref[hh:hh + 1, :BUCKETS] = idx
        vv_ref[hh:hh + 1, :BUCKETS] = m
    cp0 = pltpu.make_async_copy(iv_ref, idx_smem, sem0)
    cp1 = pltpu.make_async_copy(vv_ref, val_smem, sem1)
    cp0.start()
    cp1.start()
    cp0.wait()
    cp1.wait()

    for hh in range(G):
        ka = k_ref[0, hh].astype(jnp.bfloat16).reshape(BUCKETS, BSZ, DH)
        va = v_ref[0, hh].astype(jnp.bfloat16).reshape(BUCKETS, BSZ, DH)
        kcat_ref[hh, :, BSZ:, :] = ka
        vcat_ref[hh, :, BSZ:, :DH] = va
        vcat_ref[hh, :, :, DH:] = jnp.ones((BUCKETS, 2 * BSZ, DH),
                                           jnp.bfloat16)
    for hh in range(G):
        for u in range(BUCKETS):
            g = idx_smem[hh, u]
            s = val_smem[hh, u]
            kcat_ref[hh, u, :BSZ, :] = (
                k_ref[0, hh, pl.ds(g * BSZ, BSZ), :] * s).astype(jnp.bfloat16)
            vcat_ref[hh, u, :BSZ, :DH] = (
                v_ref[0, hh, pl.ds(g * BSZ, BSZ), :] * s).astype(jnp.bfloat16)

    for hh in range(G):
        qa = (q_ref[0, hh] * SCALE).astype(jnp.bfloat16)
        for u in range(BUCKETS):
            d = jax.lax.dot_general(
                qa[u * BSZ:(u + 1) * BSZ, :], kcat_ref[hh, u],
                (((1,), (1,)), ((), ())),
                preferred_element_type=jnp.float32)
            p_ref[hh, u] = jnp.exp(d).astype(jnp.bfloat16)

    for hh in range(G):
        for u in range(BUCKETS):
            acc = jax.lax.dot_general(
                p_ref[hh, u], vcat_ref[hh, u], (((1,), (0,)), ((), ())),
                preferred_element_type=jnp.float32)
            o_ref[hh, u] = acc[:, :DH] / acc[:, DH:DH + 1]


@jax.jit
def kernel(q, k, v, W):
    hb = H // G
    return pl.pallas_call(
        _sinkhorn_kernel,
        grid=(BH // G,),
        in_specs=[
            pl.BlockSpec((1, G, T, DH),
                         lambda i: (i // hb, jax.lax.rem(i, hb), 0, 0)),
            pl.BlockSpec((1, G, T, DH),
                         lambda i: (i // hb, jax.lax.rem(i, hb), 0, 0)),
            pl.BlockSpec((1, G, T, DH),
                         lambda i: (i // hb, jax.lax.rem(i, hb), 0, 0)),
            pl.BlockSpec((1, G, 2 * DH, BUCKETS),
                         lambda i: (0, jax.lax.rem(i, hb), 0, 0)),
        ],
        out_specs=pl.BlockSpec((G, BUCKETS, BSZ, DH), lambda i: (i, 0, 0, 0)),
        out_shape=jax.ShapeDtypeStruct((BH, BUCKETS, BSZ, DH), jnp.float32),
        scratch_shapes=[
            pltpu.VMEM((G, BUCKETS, 2 * BSZ, DH), jnp.bfloat16),
            pltpu.VMEM((G, BUCKETS, 2 * BSZ, 2 * DH), jnp.bfloat16),
            pltpu.VMEM((G, BUCKETS, BSZ, 2 * BSZ), jnp.bfloat16),
            pltpu.VMEM((8, 128), jnp.int32),
            pltpu.VMEM((8, 128), jnp.float32),
            pltpu.SMEM((8, 128), jnp.int32),
            pltpu.SMEM((8, 128), jnp.float32),
            pltpu.SemaphoreType.DMA,
            pltpu.SemaphoreType.DMA,
        ],
    )(q, k, v, W)


# speedup vs baseline: 1.2448x; 1.0227x over previous
import jax
import jax.numpy as jnp
from jax.experimental import pallas as pl
from jax.experimental.pallas import tpu as pltpu

B, H, T, DH = 2, 16, 4096, 64
BUCKETS = 64
BSZ = T // BUCKETS
BH = B * H
G = 2
SCALE = 1024.0 ** -0.5


def _sinkhorn_kernel(q_ref, k_ref, v_ref, w_ref, o_ref,
                     kcat_ref, vcat_ref, p_ref,
                     iv_ref, vv_ref, idx_smem, val_smem, sem0, sem1):
    for hh in range(G):
        qs = jnp.sum(q_ref[0, hh].reshape(BUCKETS, BSZ, DH), axis=1)
        ks = jnp.sum(k_ref[0, hh].reshape(BUCKETS, BSZ, DH), axis=1)
        x = jnp.concatenate([qs, ks], axis=1)
        r = jnp.dot(x, w_ref[0, hh], preferred_element_type=jnp.float32)
        r = jnp.maximum(r, 0.0)
        r = jax.nn.softmax(r, axis=-1)
        rt = r.T
        m = jnp.max(rt, axis=0, keepdims=True)
        row = jax.lax.broadcasted_iota(jnp.int32, (BUCKETS, BUCKETS), 0)
        idx = jnp.min(jnp.where(rt >= m, row, BUCKETS), axis=0, keepdims=True)
        iv_ref[hh:hh + 1, :BUCKETS] = idx
        vv_ref[hh:hh + 1, :BUCKETS] = m
    cp0 = pltpu.make_async_copy(iv_ref, idx_smem, sem0)
    cp1 = pltpu.make_async_copy(vv_ref, val_smem, sem1)
    cp0.start()
    cp1.start()
    cp0.wait()
    cp1.wait()

    @pl.when(pl.program_id(0) == 0)
    def _fill_ones():
        for hh in range(G):
            vcat_ref[hh, :, :, DH:] = jnp.ones((BUCKETS, 2 * BSZ, DH),
                                               jnp.bfloat16)

    for hh in range(G):
        ka = k_ref[0, hh].astype(jnp.bfloat16).reshape(BUCKETS, BSZ, DH)
        va = v_ref[0, hh].astype(jnp.bfloat16).reshape(BUCKETS, BSZ, DH)
        kcat_ref[hh, :, BSZ:, :] = ka
        vcat_ref[hh, :, BSZ:, :DH] = va
    for hh in range(G):
        for u in range(BUCKETS):
            g = idx_smem[hh, u]
            s = val_smem[hh, u]
            kcat_ref[hh, u, :BSZ, :] = (
                k_ref[0, hh, pl.ds(g * BSZ, BSZ), :] * s).astype(jnp.bfloat16)
            vcat_ref[hh, u, :BSZ, :DH] = (
                v_ref[0, hh, pl.ds(g * BSZ, BSZ), :] * s).astype(jnp.bfloat16)

    for hh in range(G):
        qa = (q_ref[0, hh] * SCALE).astype(jnp.bfloat16)
        for u in range(BUCKETS):
            d = jax.lax.dot_general(
                qa[u * BSZ:(u + 1) * BSZ, :], kcat_ref[hh, u],
                (((1,), (1,)), ((), ())),
                preferred_element_type=jnp.float32)
            p_ref[hh, u] = jnp.exp(d).astype(jnp.bfloat16)

    for hh in range(G):
        for u in range(BUCKETS):
            acc = jax.lax.dot_general(
                p_ref[hh, u], vcat_ref[hh, u], (((1,), (0,)), ((), ())),
                preferred_element_type=jnp.float32)
            o_ref[hh, u] = acc[:, :DH] / acc[:, DH:DH + 1]


@jax.jit
def kernel(q, k, v, W):
    hb = H // G
    return pl.pallas_call(
        _sinkhorn_kernel,
        grid=(BH // G,),
        in_specs=[
            pl.BlockSpec((1, G, T, DH),
                         lambda i: (i // hb, jax.lax.rem(i, hb), 0, 0)),
            pl.BlockSpec((1, G, T, DH),
                         lambda i: (i // hb, jax.lax.rem(i, hb), 0, 0)),
            pl.BlockSpec((1, G, T, DH),
                         lambda i: (i // hb, jax.lax.rem(i, hb), 0, 0)),
            pl.BlockSpec((1, G, 2 * DH, BUCKETS),
                         lambda i: (0, jax.lax.rem(i, hb), 0, 0)),
        ],
        out_specs=pl.BlockSpec((G, BUCKETS, BSZ, DH), lambda i: (i, 0, 0, 0)),
        out_shape=jax.ShapeDtypeStruct((BH, BUCKETS, BSZ, DH), jnp.float32),
        scratch_shapes=[
            pltpu.VMEM((G, BUCKETS, 2 * BSZ, DH), jnp.bfloat16),
            pltpu.VMEM((G, BUCKETS, 2 * BSZ, 2 * DH), jnp.bfloat16),
            pltpu.VMEM((G, BUCKETS, BSZ, 2 * BSZ), jnp.bfloat16),
            pltpu.VMEM((8, 128), jnp.int32),
            pltpu.VMEM((8, 128), jnp.float32),
            pltpu.SMEM((8, 128), jnp.int32),
            pltpu.SMEM((8, 128), jnp.float32),
            pltpu.SemaphoreType.DMA,
            pltpu.SemaphoreType.DMA,
        ],
    )(q, k, v, W)
